# baseline (speedup 1.0000x reference)
import functools

import jax
import jax.numpy as jnp
from jax import lax
from jax.experimental import pallas as pl
from jax.experimental.pallas import tpu as pltpu
from jax.experimental.pallas import tpu_sc as plsc

N_TOK = 2048
D = 1024
H = 4096
E = 10
KTOP = 4
EPAD = 128
NEG = -1e30

TOK_BLK = 256
NI = N_TOK // TOK_BLK
NPAIR = N_TOK * KTOP
B_BLK = 256
NBLK = NPAIR // B_BLK + E
N_PAD = NBLK * B_BLK
NBE = 48

_NC, _NS, _L = 2, 16, 16
_NW = _NC * _NS
_DHALF = D // _NC


def _routing_body(x_ref, wg_ref, pe_ref, pr_ref, pg_ref, cnt_ref, loss_ref,
                  base_ref, be_ref, carry, imp_acc):
    i = pl.program_id(0)
    logits = jnp.dot(x_ref[...], wg_ref[...], preferred_element_type=jnp.float32)
    lane = lax.broadcasted_iota(jnp.int32, (TOK_BLK, EPAD), 1)
    logits = jnp.where(lane < E, logits, NEG)

    rem = logits
    idxs, vals = [], []
    for _ in range(KTOP):
        m = jnp.max(rem, axis=1, keepdims=True)
        sel = jnp.min(jnp.where(rem == m, lane, EPAD), axis=1, keepdims=True)
        idxs.append(sel)
        vals.append(m)
        rem = jnp.where(lane == sel, NEG, rem)

    exps = [jnp.exp(v - vals[0]) for v in vals]
    denom = exps[0] + exps[1] + exps[2] + exps[3]
    gk = [ex / denom for ex in exps]

    onehots = [(lane == idxs[k]).astype(jnp.float32) for k in range(KTOP)]
    gates = (onehots[0] * gk[0] + onehots[1] * gk[1]
             + onehots[2] * gk[2] + onehots[3] * gk[3])

    t_blk = onehots[0] + onehots[1] + onehots[2] + onehots[3]
    row = lax.broadcasted_iota(jnp.int32, (TOK_BLK, TOK_BLK), 0)
    col = lax.broadcasted_iota(jnp.int32, (TOK_BLK, TOK_BLK), 1)
    tri = (col < row).astype(jnp.bfloat16)

    @pl.when(i == 0)
    def _():
        carry[...] = jnp.zeros((1, EPAD), jnp.float32)
        imp_acc[...] = jnp.zeros((1, EPAD), jnp.float32)

    cex = jnp.dot(tri, t_blk.astype(jnp.bfloat16),
                  preferred_element_type=jnp.float32) + carry[...]

    pe = jnp.zeros((TOK_BLK, EPAD), jnp.int32)
    pr = jnp.zeros((TOK_BLK, EPAD), jnp.int32)
    pg = jnp.zeros((TOK_BLK, EPAD), jnp.float32)
    for k in range(KTOP):
        rank_k = jnp.sum(jnp.where(lane == idxs[k], cex, 0.0), axis=1,
                         keepdims=True).astype(jnp.int32)
        sel_k = lane == k
        pe = jnp.where(sel_k, idxs[k], pe)
        pr = jnp.where(sel_k, rank_k, pr)
        pg = jnp.where(sel_k, gk[k], pg)
    pe_ref[...] = pe
    pr_ref[...] = pr
    pg_ref[...] = pg

    carry[...] += jnp.sum(t_blk, axis=0, keepdims=True)
    imp_acc[...] += jnp.sum(gates, axis=0, keepdims=True)
    cnt_ref[...] = carry[...]

    @pl.when(i == NI - 1)
    def _():
        emask = lane[0:1, :] < E

        def cv_sq(v):
            s = jnp.sum(jnp.where(emask, v, 0.0))
            mean = s / E
            var = jnp.sum(jnp.where(emask, (v - mean) ** 2, 0.0)) / (E - 1)
            return var / (mean * mean + 1e-10)

        loss_ref[0, 0] = (cv_sq(imp_acc[...]) + cv_sq(carry[...])) * 1e-2

        cnts = jnp.where(emask, carry[...], 0.0)
        padded = jnp.floor((cnts + (B_BLK - 1)) / B_BLK) * B_BLK
        r128 = lax.broadcasted_iota(jnp.int32, (EPAD, EPAD), 0)
        c128 = lax.broadcasted_iota(jnp.int32, (EPAD, EPAD), 1)
        tri_excl = ((r128 < c128) & (r128 < E)).astype(jnp.bfloat16)
        base_row = jnp.dot(padded.astype(jnp.bfloat16), tri_excl,
                           preferred_element_type=jnp.float32)
        base_ref[...] = base_row.astype(jnp.int32)
        cb_row = base_row / B_BLK
        cb_col = jnp.sum(jnp.where(r128 == c128,
                                   jnp.broadcast_to(cb_row, (EPAD, EPAD)),
                                   0.0), axis=1, keepdims=True)
        hits = ((c128 >= cb_col.astype(jnp.int32)) & (r128 < E))
        cnt_hits = jnp.sum(hits.astype(jnp.float32), axis=0, keepdims=True)
        be_ref[...] = jnp.clip(cnt_hits.astype(jnp.int32) - 1, 0, E - 1)


def _routing(xf, wg):
    return pl.pallas_call(
        _routing_body,
        grid=(NI,),
        in_specs=[
            pl.BlockSpec((TOK_BLK, D), lambda i: (i, 0)),
            pl.BlockSpec((D, EPAD), lambda i: (0, 0)),
        ],
        out_specs=(
            pl.BlockSpec((TOK_BLK, EPAD), lambda i: (i, 0)),
            pl.BlockSpec((TOK_BLK, EPAD), lambda i: (i, 0)),
            pl.BlockSpec((TOK_BLK, EPAD), lambda i: (i, 0)),
            pl.BlockSpec((1, EPAD), lambda i: (0, 0)),
            pl.BlockSpec((1, 1), lambda i: (0, 0), memory_space=pltpu.SMEM),
            pl.BlockSpec((1, EPAD), lambda i: (0, 0)),
            pl.BlockSpec((1, EPAD), lambda i: (0, 0)),
        ),
        out_shape=(
            jax.ShapeDtypeStruct((N_TOK, EPAD), jnp.int32),
            jax.ShapeDtypeStruct((N_TOK, EPAD), jnp.int32),
            jax.ShapeDtypeStruct((N_TOK, EPAD), jnp.float32),
            jax.ShapeDtypeStruct((1, EPAD), jnp.float32),
            jax.ShapeDtypeStruct((1, 1), jnp.float32),
            jax.ShapeDtypeStruct((1, EPAD), jnp.int32),
            jax.ShapeDtypeStruct((1, EPAD), jnp.int32),
        ),
        scratch_shapes=[
            pltpu.VMEM((1, EPAD), jnp.float32),
            pltpu.VMEM((1, EPAD), jnp.float32),
        ],
        compiler_params=pltpu.CompilerParams(
            dimension_semantics=("arbitrary",),
        ),
    )(xf, wg)


def _iota16():
    return lax.broadcasted_iota(jnp.int32, (_L,), 0)


def _sc_mesh():
    return plsc.VectorSubcoreMesh(core_axis_name="c", subcore_axis_name="s",
                                  num_cores=_NC, num_subcores=_NS)


@functools.lru_cache(maxsize=None)
def _dispatch_meta_kernel():
    return functools.partial(
        pl.kernel,
        mesh=_sc_mesh(),
        out_type=(
            jax.ShapeDtypeStruct((N_PAD,), jnp.int32),
            jax.ShapeDtypeStruct((N_PAD,), jnp.float32),
            jax.ShapeDtypeStruct((NPAIR,), jnp.int32),
        ),
        scratch_types=[
            pltpu.VMEM((_L,), jnp.int32),
            pltpu.VMEM((NPAIR,), jnp.int32),
            pltpu.VMEM((NPAIR,), jnp.int32),
            pltpu.VMEM((NPAIR,), jnp.float32),
            pltpu.VMEM((N_PAD,), jnp.int32),
            pltpu.VMEM((N_PAD,), jnp.float32),
            pltpu.VMEM((NPAIR,), jnp.int32),
        ],
        compiler_params=pltpu.CompilerParams(needs_layout_passes=False),
    )(_dispatch_meta_body)


def _dispatch_meta_body(base_hbm, pe_hbm, pr_hbm, pg_hbm,
                        stok_hbm, sgate_hbm, pos_hbm,
                        base_v, pe_v, pr_v, pg_v, st_v, sg_v, pos_v):
    wid = lax.axis_index("s") * _NC + lax.axis_index("c")

    @pl.when(wid == 0)
    def _():
        pltpu.sync_copy(base_hbm, base_v)
        pltpu.sync_copy(pe_hbm, pe_v)
        pltpu.sync_copy(pr_hbm, pr_v)
        pltpu.sync_copy(pg_hbm, pg_v)

        zi = jnp.zeros((_L,), jnp.int32)
        zf = jnp.zeros((_L,), jnp.float32)

        def init_body(i, _):
            st_v[pl.ds(i * _L, _L)] = zi
            sg_v[pl.ds(i * _L, _L)] = zf
            return 0

        lax.fori_loop(0, N_PAD // _L, init_body, 0)

        def scat_body(i, _):
            ev = pe_v[pl.ds(i * _L, _L)]
            rv = pr_v[pl.ds(i * _L, _L)]
            gv = pg_v[pl.ds(i * _L, _L)]
            pos = plsc.load_gather(base_v, [ev]) + rv
            tok = lax.shift_right_logical(_iota16() + i * _L, 2)
            pos_v[pl.ds(i * _L, _L)] = pos
            plsc.store_scatter(st_v, [pos], tok)
            plsc.store_scatter(sg_v, [pos], gv)
            return 0

        lax.fori_loop(0, NPAIR // _L, scat_body, 0)

        pltpu.sync_copy(st_v, stok_hbm)
        pltpu.sync_copy(sg_v, sgate_hbm)
        pltpu.sync_copy(pos_v, pos_hbm)


_G_CHUNK = 56
_G_PER_W = N_PAD // _NW
_G_STEPS = _G_PER_W // _G_CHUNK


@functools.lru_cache(maxsize=None)
def _sc_gather_kernel():
    return functools.partial(
        pl.kernel,
        mesh=_sc_mesh(),
        out_type=jax.ShapeDtypeStruct((N_PAD, D), jnp.float32),
        scratch_types=[
            pltpu.VMEM((_G_CHUNK,), jnp.int32),
            pltpu.VMEM((_G_CHUNK,), jnp.int32),
            pltpu.VMEM((_G_CHUNK, D), jnp.float32),
            pltpu.VMEM((_G_CHUNK, D), jnp.float32),
            pltpu.SemaphoreType.DMA,
            pltpu.SemaphoreType.DMA,
        ],
        compiler_params=pltpu.CompilerParams(needs_layout_passes=False),
    )(_sc_gather_body)


def _sc_gather_body(x_hbm, stok_hbm, xg_hbm, idx0, idx1, rows0, rows1,
                    sem0, sem1):
    wid = lax.axis_index("s") * _NC + lax.axis_index("c")
    base = wid * _G_PER_W
    idx_v = (idx0, idx1)
    rows_v = (rows0, rows1)
    sems = (sem0, sem1)

    cps = [None, None]
    for ci in range(_G_STEPS):
        b = ci % 2
        r0 = base + ci * _G_CHUNK
        if cps[b] is not None:
            cps[b].wait()
            pltpu.sync_copy(rows_v[b],
                            xg_hbm.at[pl.ds(base + (ci - 2) * _G_CHUNK,
                                            _G_CHUNK)])
        pltpu.sync_copy(stok_hbm.at[pl.ds(r0, _G_CHUNK)], idx_v[b])
        cps[b] = pltpu.async_copy(x_hbm.at[idx_v[b]], rows_v[b], sems[b])
    for ci in range(_G_STEPS - 2, _G_STEPS):
        b = ci % 2
        cps[b].wait()
        pltpu.sync_copy(rows_v[b], xg_hbm.at[pl.ds(base + ci * _G_CHUNK,
                                                   _G_CHUNK)])


def _fc1_body(be_ref, xg_ref, w_ref, b_ref, xh_ref):
    h = jnp.dot(xg_ref[...].astype(jnp.bfloat16), w_ref[0],
                preferred_element_type=jnp.float32)
    xh_ref[...] = jnp.maximum(h + b_ref[0], 0.0).astype(jnp.bfloat16)


def _fc2_body(be_ref, xh_ref, w_ref, b_ref, g_ref, xo_ref):
    oe = jnp.dot(xh_ref[...], w_ref[0], preferred_element_type=jnp.float32)
    oe = oe + b_ref[0]
    m = jnp.max(oe, axis=1, keepdims=True)
    p = jnp.exp(oe - m)
    sm = p / jnp.sum(p, axis=1, keepdims=True)
    row = lax.broadcasted_iota(jnp.int32, (B_BLK, B_BLK), 0)
    col = lax.broadcasted_iota(jnp.int32, (B_BLK, B_BLK), 1)
    gcol = jnp.sum(jnp.where(row == col, jnp.broadcast_to(g_ref[0], (B_BLK, B_BLK)), 0.0),
                   axis=1, keepdims=True)
    xo_ref[...] = gcol * sm


def _ffn(xg, sgate2d, blk_expert, fc1w, fc1b, fc2w, fc2b):
    xh = pl.pallas_call(
        _fc1_body,
        grid_spec=pltpu.PrefetchScalarGridSpec(
            num_scalar_prefetch=1,
            grid=(NBLK,),
            in_specs=[
                pl.BlockSpec((B_BLK, D), lambda j, be: (j, 0)),
                pl.BlockSpec((1, D, H), lambda j, be: (be[j], 0, 0)),
                pl.BlockSpec((1, 1, H), lambda j, be: (be[j], 0, 0)),
            ],
            out_specs=pl.BlockSpec((B_BLK, H), lambda j, be: (j, 0)),
        ),
        out_shape=jax.ShapeDtypeStruct((N_PAD, H), jnp.bfloat16),
        compiler_params=pltpu.CompilerParams(
            dimension_semantics=("arbitrary",),
        ),
    )(blk_expert, xg, fc1w, fc1b)

    xo = pl.pallas_call(
        _fc2_body,
        grid_spec=pltpu.PrefetchScalarGridSpec(
            num_scalar_prefetch=1,
            grid=(NBLK,),
            in_specs=[
                pl.BlockSpec((B_BLK, H), lambda j, be: (j, 0)),
                pl.BlockSpec((1, H, D), lambda j, be: (be[j], 0, 0)),
                pl.BlockSpec((1, 1, D), lambda j, be: (be[j], 0, 0)),
                pl.BlockSpec((1, 1, B_BLK), lambda j, be: (j, 0, 0)),
            ],
            out_specs=pl.BlockSpec((B_BLK, D), lambda j, be: (j, 0)),
        ),
        out_shape=jax.ShapeDtypeStruct((N_PAD, D), jnp.float32),
        compiler_params=pltpu.CompilerParams(
            dimension_semantics=("arbitrary",),
        ),
    )(blk_expert, xh, fc2w, fc2b, sgate2d)
    return xo


_C_TOK_W = N_TOK // _NW
_C_SUB = 16
_C_STEPS = _C_TOK_W // _C_SUB


@functools.lru_cache(maxsize=None)
def _sc_combine_kernel():
    return functools.partial(
        pl.kernel,
        mesh=_sc_mesh(),
        out_type=jax.ShapeDtypeStruct((N_TOK, D), jnp.float32),
        scratch_types=[
            pltpu.VMEM((_C_SUB * KTOP,), jnp.int32),
            pltpu.VMEM((_C_SUB * KTOP, D), jnp.float32),
            pltpu.VMEM((_C_SUB, D), jnp.float32),
            pltpu.SemaphoreType.DMA,
        ],
        compiler_params=pltpu.CompilerParams(needs_layout_passes=False),
    )(_sc_combine_body)


def _sc_combine_body(xo_hbm, pos_hbm, y_hbm, idx_v, rows_v, out_v, sem):
    wid = lax.axis_index("s") * _NC + lax.axis_index("c")
    t0 = wid * _C_TOK_W

    def body(ci, _):
        tt = t0 + ci * _C_SUB
        pltpu.sync_copy(pos_hbm.at[pl.ds(tt * KTOP, _C_SUB * KTOP)], idx_v)
        pltpu.async_copy(xo_hbm.at[idx_v], rows_v, sem).wait()

        def acc_body(i, _):
            tl = i // (D // _L)
            cc = i % (D // _L)
            cs = pl.ds(cc * _L, _L)
            out_v[tl, cs] = (rows_v[4 * tl, cs] + rows_v[4 * tl + 1, cs]
                             + rows_v[4 * tl + 2, cs] + rows_v[4 * tl + 3, cs])
            return 0

        lax.fori_loop(0, _C_SUB * (D // _L), acc_body, 0)
        pltpu.sync_copy(out_v, y_hbm.at[pl.ds(tt, _C_SUB)])
        return 0

    lax.fori_loop(0, _C_STEPS, body, 0)


@jax.jit
def kernel(x, w_gate, fc1_w, fc1_b, fc2_w, fc2_b):
    b, l, d = x.shape
    xf = x.reshape(l, d)
    wg = jnp.zeros((D, EPAD), jnp.float32).at[:, :E].set(w_gate)

    pe, pr, pg, cnt, loss, base, be = _routing(xf, wg)

    e_pair = pe[:, :KTOP].reshape(NPAIR)
    r_pair = pr[:, :KTOP].reshape(NPAIR)
    g_pair = pg[:, :KTOP].reshape(NPAIR)
    base16 = base[0, :_L]

    stok, sgate, pos = _dispatch_meta_kernel()(base16, e_pair, r_pair, g_pair)

    xg = _sc_gather_kernel()(xf, stok)

    xo = _ffn(xg, sgate.reshape(NBLK, 1, B_BLK), be[0, :NBLK],
              fc1_w.astype(jnp.bfloat16), fc1_b.reshape(E, 1, H),
              fc2_w.astype(jnp.bfloat16), fc2_b.reshape(E, 1, D))

    y = _sc_combine_kernel()(xo, pos)

    return y.reshape(b, l, d), loss.reshape(())


# speedup vs baseline: 1.2688x; 1.2688x over previous
import functools

import jax
import jax.numpy as jnp
from jax import lax
from jax.experimental import pallas as pl
from jax.experimental.pallas import tpu as pltpu
from jax.experimental.pallas import tpu_sc as plsc

N_TOK = 2048
D = 1024
H = 4096
E = 10
KTOP = 4
EPAD = 128
NEG = -1e30

TOK_BLK = 256
NI = N_TOK // TOK_BLK
NPAIR = N_TOK * KTOP
B_BLK = 256
NBLK = NPAIR // B_BLK + E
N_PAD = NBLK * B_BLK
NBE = 48

_NC, _NS, _L = 2, 16, 16
_NW = _NC * _NS
_DHALF = D // _NC


def _routing_body(x_ref, wg_ref, pe_ref, pr_ref, pg_ref, cnt_ref, loss_ref,
                  base_ref, be_ref, carry, imp_acc):
    i = pl.program_id(0)
    logits = jnp.dot(x_ref[...], wg_ref[...], preferred_element_type=jnp.float32)
    lane = lax.broadcasted_iota(jnp.int32, (TOK_BLK, EPAD), 1)
    logits = jnp.where(lane < E, logits, NEG)

    rem = logits
    idxs, vals = [], []
    for _ in range(KTOP):
        m = jnp.max(rem, axis=1, keepdims=True)
        sel = jnp.min(jnp.where(rem == m, lane, EPAD), axis=1, keepdims=True)
        idxs.append(sel)
        vals.append(m)
        rem = jnp.where(lane == sel, NEG, rem)

    exps = [jnp.exp(v - vals[0]) for v in vals]
    denom = exps[0] + exps[1] + exps[2] + exps[3]
    gk = [ex / denom for ex in exps]

    onehots = [(lane == idxs[k]).astype(jnp.float32) for k in range(KTOP)]
    gates = (onehots[0] * gk[0] + onehots[1] * gk[1]
             + onehots[2] * gk[2] + onehots[3] * gk[3])

    t_blk = onehots[0] + onehots[1] + onehots[2] + onehots[3]
    row = lax.broadcasted_iota(jnp.int32, (TOK_BLK, TOK_BLK), 0)
    col = lax.broadcasted_iota(jnp.int32, (TOK_BLK, TOK_BLK), 1)
    tri = (col < row).astype(jnp.bfloat16)

    @pl.when(i == 0)
    def _():
        carry[...] = jnp.zeros((1, EPAD), jnp.float32)
        imp_acc[...] = jnp.zeros((1, EPAD), jnp.float32)

    cex = jnp.dot(tri, t_blk.astype(jnp.bfloat16),
                  preferred_element_type=jnp.float32) + carry[...]

    pe = jnp.zeros((TOK_BLK, EPAD), jnp.int32)
    pr = jnp.zeros((TOK_BLK, EPAD), jnp.int32)
    pg = jnp.zeros((TOK_BLK, EPAD), jnp.float32)
    for k in range(KTOP):
        rank_k = jnp.sum(jnp.where(lane == idxs[k], cex, 0.0), axis=1,
                         keepdims=True).astype(jnp.int32)
        sel_k = lane == k
        pe = jnp.where(sel_k, idxs[k], pe)
        pr = jnp.where(sel_k, rank_k, pr)
        pg = jnp.where(sel_k, gk[k], pg)
    pe_ref[...] = pe
    pr_ref[...] = pr
    pg_ref[...] = pg

    carry[...] += jnp.sum(t_blk, axis=0, keepdims=True)
    imp_acc[...] += jnp.sum(gates, axis=0, keepdims=True)
    cnt_ref[...] = carry[...]

    @pl.when(i == NI - 1)
    def _():
        emask = lane[0:1, :] < E

        def cv_sq(v):
            s = jnp.sum(jnp.where(emask, v, 0.0))
            mean = s / E
            var = jnp.sum(jnp.where(emask, (v - mean) ** 2, 0.0)) / (E - 1)
            return var / (mean * mean + 1e-10)

        loss_ref[0, 0] = (cv_sq(imp_acc[...]) + cv_sq(carry[...])) * 1e-2

        cnts = jnp.where(emask, carry[...], 0.0)
        padded = jnp.floor((cnts + (B_BLK - 1)) / B_BLK) * B_BLK
        r128 = lax.broadcasted_iota(jnp.int32, (EPAD, EPAD), 0)
        c128 = lax.broadcasted_iota(jnp.int32, (EPAD, EPAD), 1)
        tri_excl = ((r128 < c128) & (r128 < E)).astype(jnp.bfloat16)
        base_row = jnp.dot(padded.astype(jnp.bfloat16), tri_excl,
                           preferred_element_type=jnp.float32)
        base_ref[...] = base_row.astype(jnp.int32)
        cb_row = base_row / B_BLK
        cb_col = jnp.sum(jnp.where(r128 == c128,
                                   jnp.broadcast_to(cb_row, (EPAD, EPAD)),
                                   0.0), axis=1, keepdims=True)
        hits = ((c128 >= cb_col.astype(jnp.int32)) & (r128 < E))
        cnt_hits = jnp.sum(hits.astype(jnp.float32), axis=0, keepdims=True)
        be_ref[...] = jnp.clip(cnt_hits.astype(jnp.int32) - 1, 0, E - 1)


def _routing(xf, wg):
    return pl.pallas_call(
        _routing_body,
        grid=(NI,),
        in_specs=[
            pl.BlockSpec((TOK_BLK, D), lambda i: (i, 0)),
            pl.BlockSpec((D, EPAD), lambda i: (0, 0)),
        ],
        out_specs=(
            pl.BlockSpec((TOK_BLK, EPAD), lambda i: (i, 0)),
            pl.BlockSpec((TOK_BLK, EPAD), lambda i: (i, 0)),
            pl.BlockSpec((TOK_BLK, EPAD), lambda i: (i, 0)),
            pl.BlockSpec((1, EPAD), lambda i: (0, 0)),
            pl.BlockSpec((1, 1), lambda i: (0, 0), memory_space=pltpu.SMEM),
            pl.BlockSpec((1, EPAD), lambda i: (0, 0)),
            pl.BlockSpec((1, EPAD), lambda i: (0, 0)),
        ),
        out_shape=(
            jax.ShapeDtypeStruct((N_TOK, EPAD), jnp.int32),
            jax.ShapeDtypeStruct((N_TOK, EPAD), jnp.int32),
            jax.ShapeDtypeStruct((N_TOK, EPAD), jnp.float32),
            jax.ShapeDtypeStruct((1, EPAD), jnp.float32),
            jax.ShapeDtypeStruct((1, 1), jnp.float32),
            jax.ShapeDtypeStruct((1, EPAD), jnp.int32),
            jax.ShapeDtypeStruct((1, EPAD), jnp.int32),
        ),
        scratch_shapes=[
            pltpu.VMEM((1, EPAD), jnp.float32),
            pltpu.VMEM((1, EPAD), jnp.float32),
        ],
        compiler_params=pltpu.CompilerParams(
            dimension_semantics=("arbitrary",),
        ),
    )(xf, wg)


def _iota16():
    return lax.broadcasted_iota(jnp.int32, (_L,), 0)


def _sc_mesh():
    return plsc.VectorSubcoreMesh(core_axis_name="c", subcore_axis_name="s",
                                  num_cores=_NC, num_subcores=_NS)


@functools.lru_cache(maxsize=None)
def _dispatch_meta_kernel():
    return functools.partial(
        pl.kernel,
        mesh=_sc_mesh(),
        out_type=(
            jax.ShapeDtypeStruct((N_PAD,), jnp.int32),
            jax.ShapeDtypeStruct((N_PAD,), jnp.float32),
            jax.ShapeDtypeStruct((NPAIR,), jnp.int32),
        ),
        scratch_types=[
            pltpu.VMEM((_L,), jnp.int32),
            pltpu.VMEM((NPAIR,), jnp.int32),
            pltpu.VMEM((NPAIR,), jnp.int32),
            pltpu.VMEM((NPAIR,), jnp.float32),
            pltpu.VMEM((N_PAD,), jnp.int32),
            pltpu.VMEM((N_PAD,), jnp.float32),
            pltpu.VMEM((NPAIR,), jnp.int32),
        ],
        compiler_params=pltpu.CompilerParams(needs_layout_passes=False),
    )(_dispatch_meta_body)


def _dispatch_meta_body(base_hbm, pe_hbm, pr_hbm, pg_hbm,
                        stok_hbm, sgate_hbm, pos_hbm,
                        base_v, pe_v, pr_v, pg_v, st_v, sg_v, pos_v):
    wid = lax.axis_index("s") * _NC + lax.axis_index("c")

    @pl.when(wid == 0)
    def _():
        pltpu.sync_copy(base_hbm, base_v)
        pltpu.sync_copy(pe_hbm, pe_v)
        pltpu.sync_copy(pr_hbm, pr_v)
        pltpu.sync_copy(pg_hbm, pg_v)

        zi = jnp.zeros((_L,), jnp.int32)
        zf = jnp.zeros((_L,), jnp.float32)

        def init_body(i, _):
            st_v[pl.ds(i * _L, _L)] = zi
            sg_v[pl.ds(i * _L, _L)] = zf
            return 0

        lax.fori_loop(0, N_PAD // _L, init_body, 0)

        def scat_body(i, _):
            ev = pe_v[pl.ds(i * _L, _L)]
            rv = pr_v[pl.ds(i * _L, _L)]
            gv = pg_v[pl.ds(i * _L, _L)]
            pos = plsc.load_gather(base_v, [ev]) + rv
            tok = lax.shift_right_logical(_iota16() + i * _L, 2)
            pos_v[pl.ds(i * _L, _L)] = pos
            plsc.store_scatter(st_v, [pos], tok)
            plsc.store_scatter(sg_v, [pos], gv)
            return 0

        lax.fori_loop(0, NPAIR // _L, scat_body, 0)

        pltpu.sync_copy(st_v, stok_hbm)
        pltpu.sync_copy(sg_v, sgate_hbm)
        pltpu.sync_copy(pos_v, pos_hbm)


def _fc1_body(be_ref, x_ref, tok_ref, w_ref, b_ref, xh_ref):
    row = lax.broadcasted_iota(jnp.int32, (B_BLK, B_BLK), 0)
    col = lax.broadcasted_iota(jnp.int32, (B_BLK, B_BLK), 1)
    tokcol = jnp.sum(jnp.where(row == col,
                               jnp.broadcast_to(tok_ref[0], (B_BLK, B_BLK)),
                               0), axis=1, keepdims=True)
    ncol = lax.broadcasted_iota(jnp.int32, (B_BLK, N_TOK), 1)
    sel = (tokcol == ncol).astype(jnp.bfloat16)
    xg = jnp.dot(sel, x_ref[...],
                 preferred_element_type=jnp.float32).astype(jnp.bfloat16)
    h = jnp.dot(xg, w_ref[0], preferred_element_type=jnp.float32)
    xh_ref[...] = jnp.maximum(h + b_ref[0], 0.0).astype(jnp.bfloat16)


def _fc2_body(be_ref, xh_ref, w_ref, b_ref, g_ref, xo_ref):
    oe = jnp.dot(xh_ref[...], w_ref[0], preferred_element_type=jnp.float32)
    oe = oe + b_ref[0]
    m = jnp.max(oe, axis=1, keepdims=True)
    p = jnp.exp(oe - m)
    sm = p / jnp.sum(p, axis=1, keepdims=True)
    row = lax.broadcasted_iota(jnp.int32, (B_BLK, B_BLK), 0)
    col = lax.broadcasted_iota(jnp.int32, (B_BLK, B_BLK), 1)
    gcol = jnp.sum(jnp.where(row == col, jnp.broadcast_to(g_ref[0], (B_BLK, B_BLK)), 0.0),
                   axis=1, keepdims=True)
    xo_ref[...] = gcol * sm


def _ffn(xb, stok3, sgate2d, blk_expert, fc1w, fc1b, fc2w, fc2b):
    xh = pl.pallas_call(
        _fc1_body,
        grid_spec=pltpu.PrefetchScalarGridSpec(
            num_scalar_prefetch=1,
            grid=(NBLK,),
            in_specs=[
                pl.BlockSpec((N_TOK, D), lambda j, be: (0, 0)),
                pl.BlockSpec((1, 1, B_BLK), lambda j, be: (j, 0, 0)),
                pl.BlockSpec((1, D, H), lambda j, be: (be[j], 0, 0)),
                pl.BlockSpec((1, 1, H), lambda j, be: (be[j], 0, 0)),
            ],
            out_specs=pl.BlockSpec((B_BLK, H), lambda j, be: (j, 0)),
        ),
        out_shape=jax.ShapeDtypeStruct((N_PAD, H), jnp.bfloat16),
        compiler_params=pltpu.CompilerParams(
            dimension_semantics=("arbitrary",),
        ),
    )(blk_expert, xb, stok3, fc1w, fc1b)

    xo = pl.pallas_call(
        _fc2_body,
        grid_spec=pltpu.PrefetchScalarGridSpec(
            num_scalar_prefetch=1,
            grid=(NBLK,),
            in_specs=[
                pl.BlockSpec((B_BLK, H), lambda j, be: (j, 0)),
                pl.BlockSpec((1, H, D), lambda j, be: (be[j], 0, 0)),
                pl.BlockSpec((1, 1, D), lambda j, be: (be[j], 0, 0)),
                pl.BlockSpec((1, 1, B_BLK), lambda j, be: (j, 0, 0)),
            ],
            out_specs=pl.BlockSpec((B_BLK, D), lambda j, be: (j, 0)),
        ),
        out_shape=jax.ShapeDtypeStruct((N_PAD, D), jnp.float32),
        compiler_params=pltpu.CompilerParams(
            dimension_semantics=("arbitrary",),
        ),
    )(blk_expert, xh, fc2w, fc2b, sgate2d)
    return xo


_C_TOK_W = N_TOK // _NW
_C_SUB = 16
_C_STEPS = _C_TOK_W // _C_SUB


@functools.lru_cache(maxsize=None)
def _sc_combine_kernel():
    return functools.partial(
        pl.kernel,
        mesh=_sc_mesh(),
        out_type=jax.ShapeDtypeStruct((N_TOK, D), jnp.float32),
        scratch_types=[
            pltpu.VMEM((_C_SUB * KTOP,), jnp.int32),
            pltpu.VMEM((_C_SUB * KTOP, D), jnp.float32),
            pltpu.VMEM((_C_SUB, D), jnp.float32),
            pltpu.SemaphoreType.DMA,
        ],
        compiler_params=pltpu.CompilerParams(needs_layout_passes=False),
    )(_sc_combine_body)


def _sc_combine_body(xo_hbm, pos_hbm, y_hbm, idx_v, rows_v, out_v, sem):
    wid = lax.axis_index("s") * _NC + lax.axis_index("c")
    t0 = wid * _C_TOK_W

    def body(ci, _):
        tt = t0 + ci * _C_SUB
        pltpu.sync_copy(pos_hbm.at[pl.ds(tt * KTOP, _C_SUB * KTOP)], idx_v)
        pltpu.async_copy(xo_hbm.at[idx_v], rows_v, sem).wait()

        def acc_body(i, _):
            tl = i // (D // _L)
            cc = i % (D // _L)
            cs = pl.ds(cc * _L, _L)
            out_v[tl, cs] = (rows_v[4 * tl, cs] + rows_v[4 * tl + 1, cs]
                             + rows_v[4 * tl + 2, cs] + rows_v[4 * tl + 3, cs])
            return 0

        lax.fori_loop(0, _C_SUB * (D // _L), acc_body, 0)
        pltpu.sync_copy(out_v, y_hbm.at[pl.ds(tt, _C_SUB)])
        return 0

    lax.fori_loop(0, _C_STEPS, body, 0)


@jax.jit
def kernel(x, w_gate, fc1_w, fc1_b, fc2_w, fc2_b):
    b, l, d = x.shape
    xf = x.reshape(l, d)
    wg = jnp.zeros((D, EPAD), jnp.float32).at[:, :E].set(w_gate)

    pe, pr, pg, cnt, loss, base, be = _routing(xf, wg)

    e_pair = pe[:, :KTOP].reshape(NPAIR)
    r_pair = pr[:, :KTOP].reshape(NPAIR)
    g_pair = pg[:, :KTOP].reshape(NPAIR)
    base16 = base[0, :_L]

    stok, sgate, pos = _dispatch_meta_kernel()(base16, e_pair, r_pair, g_pair)

    xo = _ffn(xf.astype(jnp.bfloat16), stok.reshape(NBLK, 1, B_BLK),
              sgate.reshape(NBLK, 1, B_BLK), be[0, :NBLK],
              fc1_w.astype(jnp.bfloat16), fc1_b.reshape(E, 1, H),
              fc2_w.astype(jnp.bfloat16), fc2_b.reshape(E, 1, D))

    y = _sc_combine_kernel()(xo, pos)

    return y.reshape(b, l, d), loss.reshape(())


# speedup vs baseline: 1.4730x; 1.1610x over previous
import functools

import jax
import jax.numpy as jnp
from jax import lax
from jax.experimental import pallas as pl
from jax.experimental.pallas import tpu as pltpu
from jax.experimental.pallas import tpu_sc as plsc

N_TOK = 2048
D = 1024
H = 4096
E = 10
KTOP = 4
EPAD = 128
NEG = -1e30

TOK_BLK = 256
NI = N_TOK // TOK_BLK
NPAIR = N_TOK * KTOP
B_BLK = 256
NBLK = NPAIR // B_BLK + E
N_PAD = NBLK * B_BLK
NBE = 48

_NC, _NS, _L = 2, 16, 16
_NW = _NC * _NS
_DHALF = D // _NC


def _routing_body(x_ref, wg_ref, pe_ref, pr_ref, pg_ref, cnt_ref, loss_ref,
                  base_ref, be_ref, carry, imp_acc):
    i = pl.program_id(0)
    logits = jnp.dot(x_ref[...], wg_ref[...], preferred_element_type=jnp.float32)
    lane = lax.broadcasted_iota(jnp.int32, (TOK_BLK, EPAD), 1)
    logits = jnp.where(lane < E, logits, NEG)

    rem = logits
    idxs, vals = [], []
    for _ in range(KTOP):
        m = jnp.max(rem, axis=1, keepdims=True)
        sel = jnp.min(jnp.where(rem == m, lane, EPAD), axis=1, keepdims=True)
        idxs.append(sel)
        vals.append(m)
        rem = jnp.where(lane == sel, NEG, rem)

    exps = [jnp.exp(v - vals[0]) for v in vals]
    denom = exps[0] + exps[1] + exps[2] + exps[3]
    gk = [ex / denom for ex in exps]

    onehots = [(lane == idxs[k]).astype(jnp.float32) for k in range(KTOP)]
    gates = (onehots[0] * gk[0] + onehots[1] * gk[1]
             + onehots[2] * gk[2] + onehots[3] * gk[3])

    t_blk = onehots[0] + onehots[1] + onehots[2] + onehots[3]
    row = lax.broadcasted_iota(jnp.int32, (TOK_BLK, TOK_BLK), 0)
    col = lax.broadcasted_iota(jnp.int32, (TOK_BLK, TOK_BLK), 1)
    tri = (col < row).astype(jnp.bfloat16)

    @pl.when(i == 0)
    def _():
        carry[...] = jnp.zeros((1, EPAD), jnp.float32)
        imp_acc[...] = jnp.zeros((1, EPAD), jnp.float32)

    cex = jnp.dot(tri, t_blk.astype(jnp.bfloat16),
                  preferred_element_type=jnp.float32) + carry[...]

    pe = jnp.zeros((TOK_BLK, EPAD), jnp.int32)
    pr = jnp.zeros((TOK_BLK, EPAD), jnp.int32)
    pg = jnp.zeros((TOK_BLK, EPAD), jnp.float32)
    for k in range(KTOP):
        rank_k = jnp.sum(jnp.where(lane == idxs[k], cex, 0.0), axis=1,
                         keepdims=True).astype(jnp.int32)
        sel_k = lane == k
        pe = jnp.where(sel_k, idxs[k], pe)
        pr = jnp.where(sel_k, rank_k, pr)
        pg = jnp.where(sel_k, gk[k], pg)
    pe_ref[...] = pe
    pr_ref[...] = pr
    pg_ref[...] = pg

    carry[...] += jnp.sum(t_blk, axis=0, keepdims=True)
    imp_acc[...] += jnp.sum(gates, axis=0, keepdims=True)
    cnt_ref[...] = carry[...]

    @pl.when(i == NI - 1)
    def _():
        emask = lane[0:1, :] < E

        def cv_sq(v):
            s = jnp.sum(jnp.where(emask, v, 0.0))
            mean = s / E
            var = jnp.sum(jnp.where(emask, (v - mean) ** 2, 0.0)) / (E - 1)
            return var / (mean * mean + 1e-10)

        loss_ref[0, 0] = (cv_sq(imp_acc[...]) + cv_sq(carry[...])) * 1e-2

        cnts = jnp.where(emask, carry[...], 0.0)
        padded = jnp.floor((cnts + (B_BLK - 1)) / B_BLK) * B_BLK
        r128 = lax.broadcasted_iota(jnp.int32, (EPAD, EPAD), 0)
        c128 = lax.broadcasted_iota(jnp.int32, (EPAD, EPAD), 1)
        tri_excl = ((r128 < c128) & (r128 < E)).astype(jnp.bfloat16)
        base_row = jnp.dot(padded.astype(jnp.bfloat16), tri_excl,
                           preferred_element_type=jnp.float32)
        base_ref[...] = base_row.astype(jnp.int32)
        cb_row = base_row / B_BLK
        cb_col = jnp.sum(jnp.where(r128 == c128,
                                   jnp.broadcast_to(cb_row, (EPAD, EPAD)),
                                   0.0), axis=1, keepdims=True)
        hits = ((c128 >= cb_col.astype(jnp.int32)) & (r128 < E))
        cnt_hits = jnp.sum(hits.astype(jnp.float32), axis=0, keepdims=True)
        be_ref[...] = jnp.clip(cnt_hits.astype(jnp.int32) - 1, 0, E - 1)


def _routing(xf, wg):
    return pl.pallas_call(
        _routing_body,
        grid=(NI,),
        in_specs=[
            pl.BlockSpec((TOK_BLK, D), lambda i: (i, 0)),
            pl.BlockSpec((D, EPAD), lambda i: (0, 0)),
        ],
        out_specs=(
            pl.BlockSpec((TOK_BLK, EPAD), lambda i: (i, 0)),
            pl.BlockSpec((TOK_BLK, EPAD), lambda i: (i, 0)),
            pl.BlockSpec((TOK_BLK, EPAD), lambda i: (i, 0)),
            pl.BlockSpec((1, EPAD), lambda i: (0, 0)),
            pl.BlockSpec((1, 1), lambda i: (0, 0), memory_space=pltpu.SMEM),
            pl.BlockSpec((1, EPAD), lambda i: (0, 0)),
            pl.BlockSpec((1, EPAD), lambda i: (0, 0)),
        ),
        out_shape=(
            jax.ShapeDtypeStruct((N_TOK, EPAD), jnp.int32),
            jax.ShapeDtypeStruct((N_TOK, EPAD), jnp.int32),
            jax.ShapeDtypeStruct((N_TOK, EPAD), jnp.float32),
            jax.ShapeDtypeStruct((1, EPAD), jnp.float32),
            jax.ShapeDtypeStruct((1, 1), jnp.float32),
            jax.ShapeDtypeStruct((1, EPAD), jnp.int32),
            jax.ShapeDtypeStruct((1, EPAD), jnp.int32),
        ),
        scratch_shapes=[
            pltpu.VMEM((1, EPAD), jnp.float32),
            pltpu.VMEM((1, EPAD), jnp.float32),
        ],
        compiler_params=pltpu.CompilerParams(
            dimension_semantics=("arbitrary",),
        ),
    )(xf, wg)


def _iota16():
    return lax.broadcasted_iota(jnp.int32, (_L,), 0)


def _sc_mesh():
    return plsc.VectorSubcoreMesh(core_axis_name="c", subcore_axis_name="s",
                                  num_cores=_NC, num_subcores=_NS)


@functools.lru_cache(maxsize=None)
def _dispatch_meta_kernel():
    return functools.partial(
        pl.kernel,
        mesh=_sc_mesh(),
        out_type=(
            jax.ShapeDtypeStruct((N_PAD,), jnp.int32),
            jax.ShapeDtypeStruct((N_PAD,), jnp.float32),
            jax.ShapeDtypeStruct((NPAIR,), jnp.int32),
        ),
        scratch_types=[
            pltpu.VMEM((_L,), jnp.int32),
            pltpu.VMEM((NPAIR,), jnp.int32),
            pltpu.VMEM((NPAIR,), jnp.int32),
            pltpu.VMEM((NPAIR,), jnp.float32),
            pltpu.VMEM((N_PAD,), jnp.int32),
            pltpu.VMEM((N_PAD,), jnp.float32),
            pltpu.VMEM((NPAIR,), jnp.int32),
        ],
        compiler_params=pltpu.CompilerParams(needs_layout_passes=False),
    )(_dispatch_meta_body)


def _dispatch_meta_body(base_hbm, pe_hbm, pr_hbm, pg_hbm,
                        stok_hbm, sgate_hbm, pos_hbm,
                        base_v, pe_v, pr_v, pg_v, st_v, sg_v, pos_v):
    wid = lax.axis_index("s") * _NC + lax.axis_index("c")

    @pl.when(wid == 0)
    def _():
        pltpu.sync_copy(base_hbm, base_v)
        pltpu.sync_copy(pe_hbm, pe_v)
        pltpu.sync_copy(pr_hbm, pr_v)
        pltpu.sync_copy(pg_hbm, pg_v)

        zi = jnp.zeros((_L,), jnp.int32)
        zf = jnp.zeros((_L,), jnp.float32)

        def init_body(i, _):
            st_v[pl.ds(i * _L, _L)] = zi
            sg_v[pl.ds(i * _L, _L)] = zf
            return 0

        lax.fori_loop(0, N_PAD // _L, init_body, 0)

        def scat_body(i, _):
            ev = pe_v[pl.ds(i * _L, _L)]
            rv = pr_v[pl.ds(i * _L, _L)]
            gv = pg_v[pl.ds(i * _L, _L)]
            pos = plsc.load_gather(base_v, [ev]) + rv
            tok = lax.shift_right_logical(_iota16() + i * _L, 2)
            pos_v[pl.ds(i * _L, _L)] = pos
            plsc.store_scatter(st_v, [pos], tok)
            plsc.store_scatter(sg_v, [pos], gv)
            return 0

        lax.fori_loop(0, NPAIR // _L, scat_body, 0)

        pltpu.sync_copy(st_v, stok_hbm)
        pltpu.sync_copy(sg_v, sgate_hbm)
        pltpu.sync_copy(pos_v, pos_hbm)


def _fc1_body(be_ref, x_ref, tok_ref, w_ref, b_ref, xh_ref, wbf):
    j = pl.program_id(0)

    @pl.when((j == 0) | (be_ref[j] != be_ref[jnp.maximum(j - 1, 0)]))
    def _():
        wbf[...] = w_ref[0].astype(jnp.bfloat16)

    row = lax.broadcasted_iota(jnp.int32, (B_BLK, B_BLK), 0)
    col = lax.broadcasted_iota(jnp.int32, (B_BLK, B_BLK), 1)
    tokcol = jnp.sum(jnp.where(row == col,
                               jnp.broadcast_to(tok_ref[0], (B_BLK, B_BLK)),
                               0), axis=1, keepdims=True)
    ncol = lax.broadcasted_iota(jnp.int32, (B_BLK, N_TOK), 1)
    sel = (tokcol == ncol).astype(jnp.bfloat16)
    xg = jnp.dot(sel, x_ref[...],
                 preferred_element_type=jnp.float32).astype(jnp.bfloat16)
    h = jnp.dot(xg, wbf[...], preferred_element_type=jnp.float32)
    xh_ref[...] = jnp.maximum(h + b_ref[0], 0.0).astype(jnp.bfloat16)


def _fc2_body(be_ref, xh_ref, w_ref, b_ref, g_ref, xo_ref, wbf):
    j = pl.program_id(0)

    @pl.when((j == 0) | (be_ref[j] != be_ref[jnp.maximum(j - 1, 0)]))
    def _():
        wbf[...] = w_ref[0].astype(jnp.bfloat16)

    oe = jnp.dot(xh_ref[...], wbf[...], preferred_element_type=jnp.float32)
    oe = oe + b_ref[0]
    m = jnp.max(oe, axis=1, keepdims=True)
    p = jnp.exp(oe - m)
    sm = p / jnp.sum(p, axis=1, keepdims=True)
    row = lax.broadcasted_iota(jnp.int32, (B_BLK, B_BLK), 0)
    col = lax.broadcasted_iota(jnp.int32, (B_BLK, B_BLK), 1)
    gcol = jnp.sum(jnp.where(row == col, jnp.broadcast_to(g_ref[0], (B_BLK, B_BLK)), 0.0),
                   axis=1, keepdims=True)
    xo_ref[...] = gcol * sm


def _ffn(xb, stok3, sgate2d, blk_expert, fc1w, fc1b, fc2w, fc2b):
    xh = pl.pallas_call(
        _fc1_body,
        grid_spec=pltpu.PrefetchScalarGridSpec(
            num_scalar_prefetch=1,
            grid=(NBLK,),
            in_specs=[
                pl.BlockSpec((N_TOK, D), lambda j, be: (0, 0)),
                pl.BlockSpec((1, 1, B_BLK), lambda j, be: (j, 0, 0)),
                pl.BlockSpec((1, D, H), lambda j, be: (be[j], 0, 0)),
                pl.BlockSpec((1, 1, H), lambda j, be: (be[j], 0, 0)),
            ],
            out_specs=pl.BlockSpec((B_BLK, H), lambda j, be: (j, 0)),
            scratch_shapes=[pltpu.VMEM((D, H), jnp.bfloat16)],
        ),
        out_shape=jax.ShapeDtypeStruct((N_PAD, H), jnp.bfloat16),
        compiler_params=pltpu.CompilerParams(
            dimension_semantics=("arbitrary",),
        ),
    )(blk_expert, xb, stok3, fc1w, fc1b)

    xo = pl.pallas_call(
        _fc2_body,
        grid_spec=pltpu.PrefetchScalarGridSpec(
            num_scalar_prefetch=1,
            grid=(NBLK,),
            in_specs=[
                pl.BlockSpec((B_BLK, H), lambda j, be: (j, 0)),
                pl.BlockSpec((1, H, D), lambda j, be: (be[j], 0, 0)),
                pl.BlockSpec((1, 1, D), lambda j, be: (be[j], 0, 0)),
                pl.BlockSpec((1, 1, B_BLK), lambda j, be: (j, 0, 0)),
            ],
            out_specs=pl.BlockSpec((B_BLK, D), lambda j, be: (j, 0)),
            scratch_shapes=[pltpu.VMEM((H, D), jnp.bfloat16)],
        ),
        out_shape=jax.ShapeDtypeStruct((N_PAD, D), jnp.float32),
        compiler_params=pltpu.CompilerParams(
            dimension_semantics=("arbitrary",),
        ),
    )(blk_expert, xh, fc2w, fc2b, sgate2d)
    return xo


_C_TOK_W = N_TOK // _NW
_C_SUB = 16
_C_STEPS = _C_TOK_W // _C_SUB


@functools.lru_cache(maxsize=None)
def _sc_combine_kernel():
    return functools.partial(
        pl.kernel,
        mesh=_sc_mesh(),
        out_type=jax.ShapeDtypeStruct((N_TOK, D), jnp.float32),
        scratch_types=[
            pltpu.VMEM((_C_SUB * KTOP,), jnp.int32),
            pltpu.VMEM((_C_SUB * KTOP, D), jnp.float32),
            pltpu.VMEM((_C_SUB, D), jnp.float32),
            pltpu.SemaphoreType.DMA,
        ],
        compiler_params=pltpu.CompilerParams(needs_layout_passes=False),
    )(_sc_combine_body)


def _sc_combine_body(xo_hbm, pos_hbm, y_hbm, idx_v, rows_v, out_v, sem):
    wid = lax.axis_index("s") * _NC + lax.axis_index("c")
    t0 = wid * _C_TOK_W

    def body(ci, _):
        tt = t0 + ci * _C_SUB
        pltpu.sync_copy(pos_hbm.at[pl.ds(tt * KTOP, _C_SUB * KTOP)], idx_v)
        pltpu.async_copy(xo_hbm.at[idx_v], rows_v, sem).wait()

        def acc_body(i, _):
            tl = i // (D // _L)
            cc = i % (D // _L)
            cs = pl.ds(cc * _L, _L)
            out_v[tl, cs] = (rows_v[4 * tl, cs] + rows_v[4 * tl + 1, cs]
                             + rows_v[4 * tl + 2, cs] + rows_v[4 * tl + 3, cs])
            return 0

        lax.fori_loop(0, _C_SUB * (D // _L), acc_body, 0)
        pltpu.sync_copy(out_v, y_hbm.at[pl.ds(tt, _C_SUB)])
        return 0

    lax.fori_loop(0, _C_STEPS, body, 0)


@jax.jit
def kernel(x, w_gate, fc1_w, fc1_b, fc2_w, fc2_b):
    b, l, d = x.shape
    xf = x.reshape(l, d)
    wg = jnp.zeros((D, EPAD), jnp.float32).at[:, :E].set(w_gate)

    pe, pr, pg, cnt, loss, base, be = _routing(xf, wg)

    e_pair = pe[:, :KTOP].reshape(NPAIR)
    r_pair = pr[:, :KTOP].reshape(NPAIR)
    g_pair = pg[:, :KTOP].reshape(NPAIR)
    base16 = base[0, :_L]

    stok, sgate, pos = _dispatch_meta_kernel()(base16, e_pair, r_pair, g_pair)

    xo = _ffn(xf.astype(jnp.bfloat16), stok.reshape(NBLK, 1, B_BLK),
              sgate.reshape(NBLK, 1, B_BLK), be[0, :NBLK],
              fc1_w, fc1_b.reshape(E, 1, H),
              fc2_w, fc2_b.reshape(E, 1, D))

    y = _sc_combine_kernel()(xo, pos)

    return y.reshape(b, l, d), loss.reshape(())


# speedup vs baseline: 1.4738x; 1.0005x over previous
import functools

import jax
import jax.numpy as jnp
from jax import lax
from jax.experimental import pallas as pl
from jax.experimental.pallas import tpu as pltpu
from jax.experimental.pallas import tpu_sc as plsc

N_TOK = 2048
D = 1024
H = 4096
E = 10
KTOP = 4
EPAD = 128
NEG = -1e30

TOK_BLK = 256
NI = N_TOK // TOK_BLK
NPAIR = N_TOK * KTOP
B_BLK = 256
NBLK = NPAIR // B_BLK + E
N_PAD = NBLK * B_BLK

_NC, _NS, _L = 2, 16, 16
_NW = _NC * _NS


def _routing_body(x_ref, wg_ref, pe_ref, pr_ref, pg_ref, cnt_ref, loss_ref,
                  base_ref, be_ref, carry, imp_acc):
    i = pl.program_id(0)
    logits = jnp.dot(x_ref[...], wg_ref[...], preferred_element_type=jnp.float32)
    lane = lax.broadcasted_iota(jnp.int32, (TOK_BLK, EPAD), 1)
    logits = jnp.where(lane < E, logits, NEG)

    rem = logits
    idxs, vals = [], []
    for _ in range(KTOP):
        m = jnp.max(rem, axis=1, keepdims=True)
        sel = jnp.min(jnp.where(rem == m, lane, EPAD), axis=1, keepdims=True)
        idxs.append(sel)
        vals.append(m)
        rem = jnp.where(lane == sel, NEG, rem)

    exps = [jnp.exp(v - vals[0]) for v in vals]
    denom = exps[0] + exps[1] + exps[2] + exps[3]
    gk = [ex / denom for ex in exps]

    onehots = [(lane == idxs[k]).astype(jnp.float32) for k in range(KTOP)]
    gates = (onehots[0] * gk[0] + onehots[1] * gk[1]
             + onehots[2] * gk[2] + onehots[3] * gk[3])

    t_blk = onehots[0] + onehots[1] + onehots[2] + onehots[3]
    row = lax.broadcasted_iota(jnp.int32, (TOK_BLK, TOK_BLK), 0)
    col = lax.broadcasted_iota(jnp.int32, (TOK_BLK, TOK_BLK), 1)
    tri = (col < row).astype(jnp.bfloat16)

    @pl.when(i == 0)
    def _():
        carry[...] = jnp.zeros((1, EPAD), jnp.float32)
        imp_acc[...] = jnp.zeros((1, EPAD), jnp.float32)

    cex = jnp.dot(tri, t_blk.astype(jnp.bfloat16),
                  preferred_element_type=jnp.float32) + carry[...]

    pe = jnp.zeros((TOK_BLK, EPAD), jnp.int32)
    pr = jnp.zeros((TOK_BLK, EPAD), jnp.int32)
    pg = jnp.zeros((TOK_BLK, EPAD), jnp.float32)
    for k in range(KTOP):
        rank_k = jnp.sum(jnp.where(lane == idxs[k], cex, 0.0), axis=1,
                         keepdims=True).astype(jnp.int32)
        sel_k = lane == k
        pe = jnp.where(sel_k, idxs[k], pe)
        pr = jnp.where(sel_k, rank_k, pr)
        pg = jnp.where(sel_k, gk[k], pg)
    pe_ref[...] = pe
    pr_ref[...] = pr
    pg_ref[...] = pg

    carry[...] += jnp.sum(t_blk, axis=0, keepdims=True)
    imp_acc[...] += jnp.sum(gates, axis=0, keepdims=True)
    cnt_ref[...] = carry[...]

    @pl.when(i == NI - 1)
    def _():
        emask = lane[0:1, :] < E

        def cv_sq(v):
            s = jnp.sum(jnp.where(emask, v, 0.0))
            mean = s / E
            var = jnp.sum(jnp.where(emask, (v - mean) ** 2, 0.0)) / (E - 1)
            return var / (mean * mean + 1e-10)

        loss_ref[0, 0] = (cv_sq(imp_acc[...]) + cv_sq(carry[...])) * 1e-2

        cnts = jnp.where(emask, carry[...], 0.0)
        padded = jnp.floor((cnts + (B_BLK - 1)) / B_BLK) * B_BLK
        r128 = lax.broadcasted_iota(jnp.int32, (EPAD, EPAD), 0)
        c128 = lax.broadcasted_iota(jnp.int32, (EPAD, EPAD), 1)
        tri_excl = ((r128 < c128) & (r128 < E)).astype(jnp.bfloat16)
        base_row = jnp.dot(padded.astype(jnp.bfloat16), tri_excl,
                           preferred_element_type=jnp.float32)
        base_ref[...] = base_row.astype(jnp.int32)
        cb_row = base_row / B_BLK
        cb_col = jnp.sum(jnp.where(r128 == c128,
                                   jnp.broadcast_to(cb_row, (EPAD, EPAD)),
                                   0.0), axis=1, keepdims=True)
        hits = ((c128 >= cb_col.astype(jnp.int32)) & (r128 < E))
        cnt_hits = jnp.sum(hits.astype(jnp.float32), axis=0, keepdims=True)
        be_ref[...] = jnp.clip(cnt_hits.astype(jnp.int32) - 1, 0, E - 1)


def _routing(xf, wg):
    return pl.pallas_call(
        _routing_body,
        grid=(NI,),
        in_specs=[
            pl.BlockSpec((TOK_BLK, D), lambda i: (i, 0)),
            pl.BlockSpec((D, EPAD), lambda i: (0, 0)),
        ],
        out_specs=(
            pl.BlockSpec((TOK_BLK, EPAD), lambda i: (i, 0)),
            pl.BlockSpec((TOK_BLK, EPAD), lambda i: (i, 0)),
            pl.BlockSpec((TOK_BLK, EPAD), lambda i: (i, 0)),
            pl.BlockSpec((1, EPAD), lambda i: (0, 0)),
            pl.BlockSpec((1, 1), lambda i: (0, 0), memory_space=pltpu.SMEM),
            pl.BlockSpec((1, EPAD), lambda i: (0, 0)),
            pl.BlockSpec((1, EPAD), lambda i: (0, 0)),
        ),
        out_shape=(
            jax.ShapeDtypeStruct((N_TOK, EPAD), jnp.int32),
            jax.ShapeDtypeStruct((N_TOK, EPAD), jnp.int32),
            jax.ShapeDtypeStruct((N_TOK, EPAD), jnp.float32),
            jax.ShapeDtypeStruct((1, EPAD), jnp.float32),
            jax.ShapeDtypeStruct((1, 1), jnp.float32),
            jax.ShapeDtypeStruct((1, EPAD), jnp.int32),
            jax.ShapeDtypeStruct((1, EPAD), jnp.int32),
        ),
        scratch_shapes=[
            pltpu.VMEM((1, EPAD), jnp.float32),
            pltpu.VMEM((1, EPAD), jnp.float32),
        ],
        compiler_params=pltpu.CompilerParams(
            dimension_semantics=("arbitrary",),
        ),
    )(xf, wg)


def _iota16():
    return lax.broadcasted_iota(jnp.int32, (_L,), 0)


def _sc_mesh():
    return plsc.VectorSubcoreMesh(core_axis_name="c", subcore_axis_name="s",
                                  num_cores=_NC, num_subcores=_NS)


@functools.lru_cache(maxsize=None)
def _dispatch_meta_kernel():
    return functools.partial(
        pl.kernel,
        mesh=_sc_mesh(),
        out_type=(
            jax.ShapeDtypeStruct((N_PAD,), jnp.int32),
            jax.ShapeDtypeStruct((N_PAD,), jnp.float32),
            jax.ShapeDtypeStruct((NPAIR,), jnp.int32),
        ),
        scratch_types=[
            pltpu.VMEM((_L,), jnp.int32),
            pltpu.VMEM((NPAIR,), jnp.int32),
            pltpu.VMEM((NPAIR,), jnp.int32),
            pltpu.VMEM((NPAIR,), jnp.float32),
            pltpu.VMEM((N_PAD,), jnp.int32),
            pltpu.VMEM((N_PAD,), jnp.float32),
            pltpu.VMEM((NPAIR,), jnp.int32),
        ],
        compiler_params=pltpu.CompilerParams(needs_layout_passes=False),
    )(_dispatch_meta_body)


def _dispatch_meta_body(base_hbm, pe_hbm, pr_hbm, pg_hbm,
                        stok_hbm, sgate_hbm, pos_hbm,
                        base_v, pe_v, pr_v, pg_v, st_v, sg_v, pos_v):
    wid = lax.axis_index("s") * _NC + lax.axis_index("c")

    @pl.when(wid == 0)
    def _():
        pltpu.sync_copy(base_hbm, base_v)
        pltpu.sync_copy(pe_hbm, pe_v)
        pltpu.sync_copy(pr_hbm, pr_v)
        pltpu.sync_copy(pg_hbm, pg_v)

        zi = jnp.zeros((_L,), jnp.int32)
        zf = jnp.zeros((_L,), jnp.float32)

        def init_body(i, _):
            st_v[pl.ds(i * _L, _L)] = zi
            sg_v[pl.ds(i * _L, _L)] = zf
            return 0

        lax.fori_loop(0, N_PAD // _L, init_body, 0)

        def scat_body(i, _):
            ev = pe_v[pl.ds(i * _L, _L)]
            rv = pr_v[pl.ds(i * _L, _L)]
            gv = pg_v[pl.ds(i * _L, _L)]
            pos = plsc.load_gather(base_v, [ev]) + rv
            tok = lax.shift_right_logical(_iota16() + i * _L, 2)
            pos_v[pl.ds(i * _L, _L)] = pos
            plsc.store_scatter(st_v, [pos], tok)
            plsc.store_scatter(sg_v, [pos], gv)
            return 0

        lax.fori_loop(0, NPAIR // _L, scat_body, 0)

        pltpu.sync_copy(st_v, stok_hbm)
        pltpu.sync_copy(sg_v, sgate_hbm)
        pltpu.sync_copy(pos_v, pos_hbm)


def _fc1_body(be_ref, x_ref, tok_ref, w_ref, b_ref, xh_ref, wbf):
    j = pl.program_id(0)

    @pl.when((j == 0) | (be_ref[j] != be_ref[jnp.maximum(j - 1, 0)]))
    def _():
        wbf[...] = w_ref[0].astype(jnp.bfloat16)

    row = lax.broadcasted_iota(jnp.int32, (B_BLK, B_BLK), 0)
    col = lax.broadcasted_iota(jnp.int32, (B_BLK, B_BLK), 1)
    tokcol = jnp.sum(jnp.where(row == col,
                               jnp.broadcast_to(tok_ref[0], (B_BLK, B_BLK)),
                               0), axis=1, keepdims=True)
    ncol = lax.broadcasted_iota(jnp.int32, (B_BLK, N_TOK), 1)
    sel = (tokcol == ncol).astype(jnp.bfloat16)
    xg = jnp.dot(sel, x_ref[...],
                 preferred_element_type=jnp.float32).astype(jnp.bfloat16)
    h = jnp.dot(xg, wbf[...], preferred_element_type=jnp.float32)
    xh_ref[...] = jnp.maximum(h + b_ref[0], 0.0).astype(jnp.bfloat16)


def _fc2_body(be_ref, xh_ref, w_ref, b_ref, g_ref, xo_ref, wbf):
    j = pl.program_id(0)

    @pl.when((j == 0) | (be_ref[j] != be_ref[jnp.maximum(j - 1, 0)]))
    def _():
        wbf[...] = w_ref[0].astype(jnp.bfloat16)

    oe = jnp.dot(xh_ref[...], wbf[...], preferred_element_type=jnp.float32)
    oe = oe + b_ref[0]
    m = jnp.max(oe, axis=1, keepdims=True)
    p = jnp.exp(oe - m)
    sm = p / jnp.sum(p, axis=1, keepdims=True)
    row = lax.broadcasted_iota(jnp.int32, (B_BLK, B_BLK), 0)
    col = lax.broadcasted_iota(jnp.int32, (B_BLK, B_BLK), 1)
    gcol = jnp.sum(jnp.where(row == col, jnp.broadcast_to(g_ref[0], (B_BLK, B_BLK)), 0.0),
                   axis=1, keepdims=True)
    xo_ref[...] = gcol * sm


def _ffn(xb, stok3, sgate2d, blk_expert, fc1w, fc1b, fc2w, fc2b):
    xh = pl.pallas_call(
        _fc1_body,
        grid_spec=pltpu.PrefetchScalarGridSpec(
            num_scalar_prefetch=1,
            grid=(NBLK,),
            in_specs=[
                pl.BlockSpec((N_TOK, D), lambda j, be: (0, 0)),
                pl.BlockSpec((1, 1, B_BLK), lambda j, be: (j, 0, 0)),
                pl.BlockSpec((1, D, H), lambda j, be: (be[j], 0, 0)),
                pl.BlockSpec((1, 1, H), lambda j, be: (be[j], 0, 0)),
            ],
            out_specs=pl.BlockSpec((B_BLK, H), lambda j, be: (j, 0)),
            scratch_shapes=[pltpu.VMEM((D, H), jnp.bfloat16)],
        ),
        out_shape=jax.ShapeDtypeStruct((N_PAD, H), jnp.bfloat16),
        compiler_params=pltpu.CompilerParams(
            dimension_semantics=("arbitrary",),
        ),
    )(blk_expert, xb, stok3, fc1w, fc1b)

    xo = pl.pallas_call(
        _fc2_body,
        grid_spec=pltpu.PrefetchScalarGridSpec(
            num_scalar_prefetch=1,
            grid=(NBLK,),
            in_specs=[
                pl.BlockSpec((B_BLK, H), lambda j, be: (j, 0)),
                pl.BlockSpec((1, H, D), lambda j, be: (be[j], 0, 0)),
                pl.BlockSpec((1, 1, D), lambda j, be: (be[j], 0, 0)),
                pl.BlockSpec((1, 1, B_BLK), lambda j, be: (j, 0, 0)),
            ],
            out_specs=pl.BlockSpec((B_BLK, D), lambda j, be: (j, 0)),
            scratch_shapes=[pltpu.VMEM((H, D), jnp.bfloat16)],
        ),
        out_shape=jax.ShapeDtypeStruct((N_PAD, D), jnp.float32),
        compiler_params=pltpu.CompilerParams(
            dimension_semantics=("arbitrary",),
        ),
    )(blk_expert, xh, fc2w, fc2b, sgate2d)
    return xo


_C_TOK_W = N_TOK // _NW
_C_SUB = 16
_C_STEPS = _C_TOK_W // _C_SUB


@functools.lru_cache(maxsize=None)
def _sc_combine_kernel():
    return functools.partial(
        pl.kernel,
        mesh=_sc_mesh(),
        out_type=jax.ShapeDtypeStruct((N_TOK, D), jnp.float32),
        scratch_types=[
            pltpu.VMEM((_C_SUB * KTOP,), jnp.int32),
            pltpu.VMEM((_C_SUB * KTOP, D), jnp.float32),
            pltpu.VMEM((_C_SUB, D), jnp.float32),
            pltpu.SemaphoreType.DMA,
        ],
        compiler_params=pltpu.CompilerParams(needs_layout_passes=False),
    )(_sc_combine_body)


def _sc_combine_body(xo_hbm, pos_hbm, y_hbm, idx_v, rows_v, out_v, sem):
    wid = lax.axis_index("s") * _NC + lax.axis_index("c")
    t0 = wid * _C_TOK_W

    def body(ci, _):
        tt = t0 + ci * _C_SUB
        pltpu.sync_copy(pos_hbm.at[pl.ds(tt * KTOP, _C_SUB * KTOP)], idx_v)
        pltpu.async_copy(xo_hbm.at[idx_v], rows_v, sem).wait()

        def acc_body(i, _):
            tl = i // (D // _L)
            cc = i % (D // _L)
            cs = pl.ds(cc * _L, _L)
            out_v[tl, cs] = (rows_v[4 * tl, cs] + rows_v[4 * tl + 1, cs]
                             + rows_v[4 * tl + 2, cs] + rows_v[4 * tl + 3, cs])
            return 0

        lax.fori_loop(0, _C_SUB * (D // _L), acc_body, 0)
        pltpu.sync_copy(out_v, y_hbm.at[pl.ds(tt, _C_SUB)])
        return 0

    lax.fori_loop(0, _C_STEPS, body, 0)


@jax.jit
def kernel(x, w_gate, fc1_w, fc1_b, fc2_w, fc2_b):
    b, l, d = x.shape
    xf = x.reshape(l, d)
    wg = jnp.zeros((D, EPAD), jnp.float32).at[:, :E].set(w_gate)

    pe, pr, pg, cnt, loss, base, be = _routing(xf, wg)

    e_pair = pe[:, :KTOP].reshape(NPAIR)
    r_pair = pr[:, :KTOP].reshape(NPAIR)
    g_pair = pg[:, :KTOP].reshape(NPAIR)
    base16 = base[0, :_L]

    stok, sgate, pos = _dispatch_meta_kernel()(base16, e_pair, r_pair, g_pair)

    xo = _ffn(xf.astype(jnp.bfloat16), stok.reshape(NBLK, 1, B_BLK),
              sgate.reshape(NBLK, 1, B_BLK), be[0, :NBLK],
              fc1_w, fc1_b.reshape(E, 1, H),
              fc2_w, fc2_b.reshape(E, 1, D))

    y = _sc_combine_kernel()(xo, pos)

    return y.reshape(b, l, d), loss.reshape(())
